# baseline (speedup 1.0000x reference)
import jax
import jax.numpy as jnp
from jax.experimental import pallas as pl
from jax.experimental.pallas import tpu as pltpu

_N = 10000
_D = 256
_C = 64

_TM = 80
_NBUF = 3
_NSLICE = 1
_TS = _TM // _NSLICE
_NBLK = _N // _TM


def _gcn_kernel(x_hbm, adj_hbm, w1_ref, b1_ref, wmt_ref, bm_ref, out_ref,
                abuf, xbuf, sup, sem, xsem):
    i = pl.program_id(0)

    def slice_copy(blk, s):
        return pltpu.make_async_copy(
            adj_hbm.at[pl.ds(blk * _TM + s * _TS, _TS), :],
            abuf.at[blk % _NBUF, pl.ds(s * _TS, _TS), :],
            sem.at[blk % _NBUF, s],
        )

    def x_copy():
        return pltpu.make_async_copy(x_hbm, xbuf, xsem)

    @pl.when(i == 0)
    def _():
        x_copy().start()
        for blk in range(_NBUF - 1):
            for s in range(_NSLICE):
                slice_copy(blk, s).start()
        x_copy().wait()
        sup[...] = jnp.dot(xbuf[...], w1_ref[...],
                           preferred_element_type=jnp.float32
                           ).astype(jnp.bfloat16)

    @pl.when((i >= 1) & (i + _NBUF - 2 < _NBLK))
    def _():
        for s in range(_NSLICE):
            slice_copy(i + _NBUF - 2, s).start()

    @pl.when(i >= 1)
    def _():
        b = i - 1
        for s in range(_NSLICE):
            slice_copy(b, s).wait()
        h = jnp.dot(abuf[b % _NBUF].astype(jnp.bfloat16), sup[...],
                    preferred_element_type=jnp.float32)
        h = jnp.maximum(h + b1_ref[...], 0.0)
        out_ref[...] = jnp.dot(
            h, wmt_ref[...], preferred_element_type=jnp.float32,
        ) + bm_ref[...]


def kernel(x, adj, W1, b1, W_mlp, b_mlp):
    wmt = W_mlp.T
    b1_2d = b1.reshape(1, _D)
    bm_2d = b_mlp.reshape(1, _C)

    out = pl.pallas_call(
        _gcn_kernel,
        grid=(_NBLK + 1,),
        in_specs=[
            pl.BlockSpec(memory_space=pl.ANY),
            pl.BlockSpec(memory_space=pl.ANY),
            pl.BlockSpec((_D, _D), lambda i: (0, 0)),
            pl.BlockSpec((1, _D), lambda i: (0, 0)),
            pl.BlockSpec((_D, _C), lambda i: (0, 0)),
            pl.BlockSpec((1, _C), lambda i: (0, 0)),
        ],
        out_specs=pl.BlockSpec(
            (_TM, _C), lambda i: (jnp.maximum(i - 1, 0), 0)),
        out_shape=jax.ShapeDtypeStruct((_N, _C), jnp.float32),
        scratch_shapes=[
            pltpu.VMEM((_NBUF, _TM, _N), jnp.float32),
            pltpu.VMEM((_N, _D), jnp.float32),
            pltpu.VMEM((_N, _D), jnp.bfloat16),
            pltpu.SemaphoreType.DMA((_NBUF, _NSLICE)),
            pltpu.SemaphoreType.DMA,
        ],
        compiler_params=pltpu.CompilerParams(
            dimension_semantics=("arbitrary",),
            vmem_limit_bytes=100 * 1024 * 1024,
        ),
    )(x, adj, W1, b1_2d, wmt, bm_2d)
    return out


# speedup vs baseline: 1.2024x; 1.2024x over previous
import jax
import jax.numpy as jnp
from jax.experimental import pallas as pl
from jax.experimental.pallas import tpu as pltpu

_N = 10000
_D = 256
_C = 64

_NCORE = 2
_TM = 200
_NBUF = 3
_NROW = _N // _NCORE
_NBLK = _NROW // _TM


def _gcn_kernel(x_hbm, adj_hbm, w1_ref, b1_ref, wmt_ref, bm_ref, out_ref,
                abuf, xbuf, sup, sem, xsem):
    c = pl.program_id(0)
    i = pl.program_id(1)
    row0 = c * _NROW

    def block_copy(blk):
        return pltpu.make_async_copy(
            adj_hbm.at[pl.ds(row0 + blk * _TM, _TM), :],
            abuf.at[blk % _NBUF],
            sem.at[blk % _NBUF],
        )

    def x_copy():
        return pltpu.make_async_copy(x_hbm, xbuf, xsem)

    @pl.when(i == 0)
    def _():
        x_copy().start()
        for blk in range(_NBUF - 1):
            block_copy(blk).start()
        x_copy().wait()
        sup[...] = jnp.dot(xbuf[...], w1_ref[...],
                           preferred_element_type=jnp.float32
                           ).astype(jnp.bfloat16)

    @pl.when((i >= 1) & (i + _NBUF - 2 < _NBLK))
    def _():
        block_copy(i + _NBUF - 2).start()

    @pl.when(i >= 1)
    def _():
        b = i - 1
        block_copy(b).wait()
        h = jnp.dot(abuf[b % _NBUF].astype(jnp.bfloat16), sup[...],
                    preferred_element_type=jnp.float32)
        h = jnp.maximum(h + b1_ref[...], 0.0)
        out_ref[...] = jnp.dot(
            h, wmt_ref[...], preferred_element_type=jnp.float32,
        ) + bm_ref[...]


def kernel(x, adj, W1, b1, W_mlp, b_mlp):
    wmt = W_mlp.T
    b1_2d = b1.reshape(1, _D)
    bm_2d = b_mlp.reshape(1, _C)

    out = pl.pallas_call(
        _gcn_kernel,
        grid=(_NCORE, _NBLK + 1),
        in_specs=[
            pl.BlockSpec(memory_space=pl.ANY),
            pl.BlockSpec(memory_space=pl.ANY),
            pl.BlockSpec((_D, _D), lambda c, i: (0, 0)),
            pl.BlockSpec((1, _D), lambda c, i: (0, 0)),
            pl.BlockSpec((_D, _C), lambda c, i: (0, 0)),
            pl.BlockSpec((1, _C), lambda c, i: (0, 0)),
        ],
        out_specs=pl.BlockSpec(
            (_TM, _C),
            lambda c, i: (c * _NBLK + jnp.maximum(i - 1, 0), 0)),
        out_shape=jax.ShapeDtypeStruct((_N, _C), jnp.float32),
        scratch_shapes=[
            pltpu.VMEM((_NBUF, _TM, _N), jnp.float32),
            pltpu.VMEM((_N, _D), jnp.float32),
            pltpu.VMEM((_N, _D), jnp.bfloat16),
            pltpu.SemaphoreType.DMA((_NBUF,)),
            pltpu.SemaphoreType.DMA,
        ],
        compiler_params=pltpu.CompilerParams(
            dimension_semantics=("parallel", "arbitrary"),
            vmem_limit_bytes=100 * 1024 * 1024,
        ),
    )(x, adj, W1, b1_2d, wmt, bm_2d)
    return out


# speedup vs baseline: 1.2337x; 1.0261x over previous
import jax
import jax.numpy as jnp
from jax.experimental import pallas as pl
from jax.experimental.pallas import tpu as pltpu

_N = 10000
_D = 256
_C = 64

_TM = 200
_NBUF = 4
_NSLICE = 5
_TS = _TM // _NSLICE
_NBLK = _N // _TM


def _gcn_kernel(x_hbm, adj_hbm, w1_ref, b1_ref, wmt_ref, bm_ref, out_ref,
                abuf, xbuf, sup, sem, xsem):
    i = pl.program_id(0)

    def slice_copy(blk, s):
        return pltpu.make_async_copy(
            adj_hbm.at[pl.ds(blk * _TM + s * _TS, _TS), :],
            abuf.at[blk % _NBUF, pl.ds(s * _TS, _TS), :],
            sem.at[blk % _NBUF, s],
        )

    def start_block(blk):
        for s in range(_NSLICE):
            slice_copy(blk, s).start()

    def wait_block(blk):
        for s in range(_NSLICE):
            slice_copy(blk, s).wait()

    def x_copy():
        return pltpu.make_async_copy(x_hbm, xbuf, xsem)

    @pl.when(i == 0)
    def _():
        x_copy().start()
        for blk in range(_NBUF - 1):
            start_block(blk)
        x_copy().wait()
        sup[...] = jnp.dot(xbuf[...], w1_ref[...],
                           preferred_element_type=jnp.float32)

    @pl.when((i >= 1) & (i + _NBUF - 2 < _NBLK))
    def _():
        start_block(i + _NBUF - 2)

    @pl.when(i >= 1)
    def _():
        b = i - 1
        wait_block(b)
        h = jnp.dot(abuf[b % _NBUF], sup[...],
                    preferred_element_type=jnp.float32)
        h = jnp.maximum(h + b1_ref[...], 0.0)
        out_ref[...] = jnp.dot(
            h, wmt_ref[...], preferred_element_type=jnp.float32,
        ) + bm_ref[...]


def kernel(x, adj, W1, b1, W_mlp, b_mlp):
    wmt = W_mlp.T
    b1_2d = b1.reshape(1, _D)
    bm_2d = b_mlp.reshape(1, _C)

    out = pl.pallas_call(
        _gcn_kernel,
        grid=(_NBLK + 1,),
        in_specs=[
            pl.BlockSpec(memory_space=pl.ANY),
            pl.BlockSpec(memory_space=pl.ANY),
            pl.BlockSpec((_D, _D), lambda i: (0, 0)),
            pl.BlockSpec((1, _D), lambda i: (0, 0)),
            pl.BlockSpec((_D, _C), lambda i: (0, 0)),
            pl.BlockSpec((1, _C), lambda i: (0, 0)),
        ],
        out_specs=pl.BlockSpec(
            (_TM, _C), lambda i: (jnp.maximum(i - 1, 0), 0)),
        out_shape=jax.ShapeDtypeStruct((_N, _C), jnp.float32),
        scratch_shapes=[
            pltpu.VMEM((_NBUF, _TM, _N), jnp.float32),
            pltpu.VMEM((_N, _D), jnp.float32),
            pltpu.VMEM((_N, _D), jnp.float32),
            pltpu.SemaphoreType.DMA((_NBUF, _NSLICE)),
            pltpu.SemaphoreType.DMA,
        ],
        compiler_params=pltpu.CompilerParams(
            dimension_semantics=("arbitrary",),
            vmem_limit_bytes=100 * 1024 * 1024,
        ),
    )(x, adj, W1, b1_2d, wmt, bm_2d)
    return out


# speedup vs baseline: 1.2364x; 1.0022x over previous
import jax
import jax.numpy as jnp
from jax.experimental import pallas as pl
from jax.experimental.pallas import tpu as pltpu

_N = 10000
_D = 256
_C = 64

_TM = 400
_NBUF = 2
_NSLICE = 10
_TS = _TM // _NSLICE
_NBLK = _N // _TM


def _gcn_kernel(x_hbm, adj_hbm, w1_ref, b1_ref, wmt_ref, bm_ref, out_ref,
                abuf, xbuf, sup, sem, xsem):
    i = pl.program_id(0)

    def slice_copy(blk, s):
        return pltpu.make_async_copy(
            adj_hbm.at[pl.ds(blk * _TM + s * _TS, _TS), :],
            abuf.at[blk % _NBUF, pl.ds(s * _TS, _TS), :],
            sem.at[blk % _NBUF, s],
        )

    def start_block(blk):
        for s in range(_NSLICE):
            slice_copy(blk, s).start()

    def wait_block(blk):
        for s in range(_NSLICE):
            slice_copy(blk, s).wait()

    def x_copy():
        return pltpu.make_async_copy(x_hbm, xbuf, xsem)

    @pl.when(i == 0)
    def _():
        x_copy().start()
        for blk in range(_NBUF - 1):
            start_block(blk)
        x_copy().wait()
        sup[...] = jnp.dot(xbuf[...], w1_ref[...],
                           preferred_element_type=jnp.float32)

    @pl.when((i >= 1) & (i + _NBUF - 2 < _NBLK))
    def _():
        start_block(i + _NBUF - 2)

    @pl.when(i >= 1)
    def _():
        b = i - 1
        wait_block(b)
        h = jnp.dot(abuf[b % _NBUF], sup[...],
                    preferred_element_type=jnp.float32)
        h = jnp.maximum(h + b1_ref[...], 0.0)
        out_ref[...] = jnp.dot(
            h, wmt_ref[...], preferred_element_type=jnp.float32,
        ) + bm_ref[...]


def kernel(x, adj, W1, b1, W_mlp, b_mlp):
    wmt = W_mlp.T
    b1_2d = b1.reshape(1, _D)
    bm_2d = b_mlp.reshape(1, _C)

    out = pl.pallas_call(
        _gcn_kernel,
        grid=(_NBLK + 1,),
        in_specs=[
            pl.BlockSpec(memory_space=pl.ANY),
            pl.BlockSpec(memory_space=pl.ANY),
            pl.BlockSpec((_D, _D), lambda i: (0, 0)),
            pl.BlockSpec((1, _D), lambda i: (0, 0)),
            pl.BlockSpec((_D, _C), lambda i: (0, 0)),
            pl.BlockSpec((1, _C), lambda i: (0, 0)),
        ],
        out_specs=pl.BlockSpec(
            (_TM, _C), lambda i: (jnp.maximum(i - 1, 0), 0)),
        out_shape=jax.ShapeDtypeStruct((_N, _C), jnp.float32),
        scratch_shapes=[
            pltpu.VMEM((_NBUF, _TM, _N), jnp.float32),
            pltpu.VMEM((_N, _D), jnp.float32),
            pltpu.VMEM((_N, _D), jnp.float32),
            pltpu.SemaphoreType.DMA((_NBUF, _NSLICE)),
            pltpu.SemaphoreType.DMA,
        ],
        compiler_params=pltpu.CompilerParams(
            dimension_semantics=("arbitrary",),
            vmem_limit_bytes=100 * 1024 * 1024,
        ),
    )(x, adj, W1, b1_2d, wmt, bm_2d)
    return out


# speedup vs baseline: 1.2540x; 1.0143x over previous
import jax
import jax.numpy as jnp
from jax.experimental import pallas as pl
from jax.experimental.pallas import tpu as pltpu

_N = 10000
_D = 256
_C = 64

_TM = 200
_NBUF = 3
_NSLICE = 5
_TS = _TM // _NSLICE
_NBLK = _N // _TM
_XH = _N // 2


def _gcn_kernel(x_hbm, adj_hbm, w1_ref, b1_ref, wmt_ref, bm_ref, out_ref,
                abuf, xbuf, sup, sem, xsem):
    i = pl.program_id(0)

    def slice_copy(blk, s):
        return pltpu.make_async_copy(
            adj_hbm.at[pl.ds(blk * _TM + s * _TS, _TS), :],
            abuf.at[blk % _NBUF, pl.ds(s * _TS, _TS), :],
            sem.at[blk % _NBUF, s],
        )

    def start_block(blk):
        for s in range(_NSLICE):
            slice_copy(blk, s).start()

    def wait_block(blk):
        for s in range(_NSLICE):
            slice_copy(blk, s).wait()

    def x_copy(h):
        return pltpu.make_async_copy(
            x_hbm.at[pl.ds(h * _XH, _XH), :],
            xbuf.at[pl.ds(h * _XH, _XH), :],
            xsem.at[h],
        )

    @pl.when(i == 0)
    def _():
        x_copy(0).start()
        x_copy(1).start()
        for blk in range(_NBUF - 1):
            start_block(blk)
        for h in range(2):
            x_copy(h).wait()
            sup[pl.ds(h * _XH, _XH), :] = jnp.dot(
                xbuf[pl.ds(h * _XH, _XH), :], w1_ref[...],
                preferred_element_type=jnp.float32)

    @pl.when((i >= 1) & (i + _NBUF - 2 < _NBLK))
    def _():
        start_block(i + _NBUF - 2)

    @pl.when(i >= 1)
    def _():
        b = i - 1
        wait_block(b)
        h = jnp.dot(abuf[b % _NBUF], sup[...],
                    preferred_element_type=jnp.float32)
        h = jnp.maximum(h + b1_ref[...], 0.0)
        out_ref[...] = jnp.dot(
            h, wmt_ref[...], preferred_element_type=jnp.float32,
        ) + bm_ref[...]


def kernel(x, adj, W1, b1, W_mlp, b_mlp):
    wmt = W_mlp.T
    b1_2d = b1.reshape(1, _D)
    bm_2d = b_mlp.reshape(1, _C)

    out = pl.pallas_call(
        _gcn_kernel,
        grid=(_NBLK + 1,),
        in_specs=[
            pl.BlockSpec(memory_space=pl.ANY),
            pl.BlockSpec(memory_space=pl.ANY),
            pl.BlockSpec((_D, _D), lambda i: (0, 0)),
            pl.BlockSpec((1, _D), lambda i: (0, 0)),
            pl.BlockSpec((_D, _C), lambda i: (0, 0)),
            pl.BlockSpec((1, _C), lambda i: (0, 0)),
        ],
        out_specs=pl.BlockSpec(
            (_TM, _C), lambda i: (jnp.maximum(i - 1, 0), 0)),
        out_shape=jax.ShapeDtypeStruct((_N, _C), jnp.float32),
        scratch_shapes=[
            pltpu.VMEM((_NBUF, _TM, _N), jnp.float32),
            pltpu.VMEM((_N, _D), jnp.float32),
            pltpu.VMEM((_N, _D), jnp.float32),
            pltpu.SemaphoreType.DMA((_NBUF, _NSLICE)),
            pltpu.SemaphoreType.DMA((2,)),
        ],
        compiler_params=pltpu.CompilerParams(
            dimension_semantics=("arbitrary",),
            vmem_limit_bytes=100 * 1024 * 1024,
        ),
    )(x, adj, W1, b1_2d, wmt, bm_2d)
    return out


# speedup vs baseline: 1.2667x; 1.0101x over previous
import jax
import jax.numpy as jnp
from jax.experimental import pallas as pl
from jax.experimental.pallas import tpu as pltpu

_N = 10000
_D = 256
_C = 64

_TM = 400
_NBLK = _N // _TM


def _gcn_kernel(x_ref, adj_ref, w1_ref, b1_ref, wmt_ref, bm_ref, out_ref,
                sup):
    i = pl.program_id(0)

    @pl.when(i == 0)
    def _():
        sup[...] = jnp.dot(x_ref[...], w1_ref[...],
                           preferred_element_type=jnp.float32)

    h = jnp.dot(adj_ref[...], sup[...],
                preferred_element_type=jnp.float32)
    h = jnp.maximum(h + b1_ref[...], 0.0)
    out_ref[...] = jnp.dot(
        h, wmt_ref[...], preferred_element_type=jnp.float32,
    ) + bm_ref[...]


def kernel(x, adj, W1, b1, W_mlp, b_mlp):
    wmt = W_mlp.T
    b1_2d = b1.reshape(1, _D)
    bm_2d = b_mlp.reshape(1, _C)

    out = pl.pallas_call(
        _gcn_kernel,
        grid=(_NBLK,),
        in_specs=[
            pl.BlockSpec((_N, _D), lambda i: (0, 0)),
            pl.BlockSpec((_TM, _N), lambda i: (i, 0)),
            pl.BlockSpec((_D, _D), lambda i: (0, 0)),
            pl.BlockSpec((1, _D), lambda i: (0, 0)),
            pl.BlockSpec((_D, _C), lambda i: (0, 0)),
            pl.BlockSpec((1, _C), lambda i: (0, 0)),
        ],
        out_specs=pl.BlockSpec((_TM, _C), lambda i: (i, 0)),
        out_shape=jax.ShapeDtypeStruct((_N, _C), jnp.float32),
        scratch_shapes=[
            pltpu.VMEM((_N, _D), jnp.float32),
        ],
        compiler_params=pltpu.CompilerParams(
            dimension_semantics=("arbitrary",),
            vmem_limit_bytes=100 * 1024 * 1024,
        ),
    )(x, adj, W1, b1_2d, wmt, bm_2d)
    return out
